# Initial kernel scaffold; baseline (speedup 1.0000x reference)
import jax
import jax.numpy as jnp
from jax.experimental import pallas as pl


def kernel(x, emb):
    raise NotImplementedError("write your pallas kernel here")


# speedup vs baseline: 5.1260x; 5.1260x over previous
import functools

import jax
import jax.numpy as jnp
from jax import lax
from jax.experimental import pallas as pl
from jax.experimental.pallas import tpu as pltpu
from jax.experimental.pallas import tpu_sc as plsc

_CHUNKS = 100000


def _idx_body(x_ref, idx_ref):
    v = (jnp.tanh(x_ref[...]) + 1.0) * (float(_CHUNKS) / 2.0)
    idx_ref[...] = jnp.minimum(v.astype(jnp.int32), _CHUNKS - 1)


def _compute_idx(x):
    return pl.pallas_call(
        _idx_body,
        out_shape=jax.ShapeDtypeStruct(x.shape, jnp.int32),
    )(x)


@functools.cache
def _make_gather(B, D):
    info = plsc.get_sparse_core_info()
    nc, ns = info.num_cores, info.num_subcores
    nw = nc * ns
    b_per_w = B // nw
    CH = 128
    n_ch = b_per_w // CH
    NBUF = 4
    LOOK = 2
    assert nw * b_per_w == B and CH * n_ch == b_per_w
    assert n_ch % NBUF == 0 and n_ch >= NBUF + LOOK
    n_groups = n_ch // NBUF
    mesh = plsc.VectorSubcoreMesh(core_axis_name="c", subcore_axis_name="s")

    @functools.partial(
        pl.kernel,
        mesh=mesh,
        out_type=jax.ShapeDtypeStruct((B, D), jnp.float32),
        scratch_types=[
            pltpu.VMEM((n_ch, CH), jnp.int32),
            pltpu.VMEM((NBUF, CH, D), jnp.float32),
        ]
        + [pltpu.SemaphoreType.DMA] * (2 * NBUF),
        compiler_params=pltpu.CompilerParams(use_tc_tiling_on_sc=False),
    )
    def gather(emb_hbm, idx_hbm, out_hbm, idx_v, rows_v, *sems):
        G = sems[:NBUF]
        W = sems[NBUF:]
        wid = lax.axis_index("s") * nc + lax.axis_index("c")
        base = wid * b_per_w
        pltpu.sync_copy(idx_hbm.at[wid], idx_v)

        def chunk_step(c, j, fire_pre, wait_wb):
            if fire_pre:
                j2 = (j + LOOK) % NBUF
                if wait_wb:
                    pltpu.make_async_copy(
                        rows_v.at[j2], out_hbm.at[pl.ds(base, CH)], W[j2]
                    ).wait()
                pltpu.make_async_copy(
                    emb_hbm.at[idx_v.at[c + LOOK]], rows_v.at[j2], G[j2]
                ).start()
            pltpu.make_async_copy(
                emb_hbm.at[idx_v.at[c]], rows_v.at[j], G[j]
            ).wait()
            pltpu.make_async_copy(
                rows_v.at[j], out_hbm.at[pl.ds(base + c * CH, CH)], W[j]
            ).start()

        for c in range(LOOK):
            pltpu.make_async_copy(
                emb_hbm.at[idx_v.at[c]], rows_v.at[c], G[c]
            ).start()
        for j in range(NBUF):
            chunk_step(j, j, fire_pre=j + LOOK < n_ch, wait_wb=j + LOOK >= NBUF)

        def group(g, carry):
            c0 = g * NBUF
            for j in range(NBUF):
                chunk_step(c0 + j, j, fire_pre=True, wait_wb=True)
            return carry

        lax.fori_loop(1, n_groups - 1, group, 0)

        for j in range(NBUF):
            c = n_ch - NBUF + j
            chunk_step(c, j, fire_pre=c + LOOK < n_ch, wait_wb=c + LOOK < n_ch)
        for c in range(n_ch - NBUF, n_ch):
            pltpu.make_async_copy(
                rows_v.at[c % NBUF], out_hbm.at[pl.ds(base, CH)], W[c % NBUF]
            ).wait()

    return gather


def kernel(x, emb):
    B = x.shape[0] * x.shape[1]
    D = emb.shape[1]
    idx = _compute_idx(x).reshape(32, -1, 128)
    out = _make_gather(B, D)(emb, idx)
    return out.reshape(x.shape[0], x.shape[1], D)
